# baseline (speedup 1.0000x reference)
import functools

import jax
import jax.numpy as jnp
from jax import lax
from jax.experimental import pallas as pl
from jax.experimental.pallas import tpu as pltpu
import jax.experimental.pallas.tpu_sc as plsc

N = 10000
E = 320000
D = 128
DE = 16
DH = D // 2

NC = 2
NS = 16
EPW = E // NS
C = 80
NCHUNK = EPW // C
SG = 10
NG = NCHUNK // SG
NP = 10240
RPT = NP // NS


BE = 2000


def _edge_mlp_body(ef_ref, w1_ref, b1_ref, w2_ref, b2_ref, out_ref):
    hid = jnp.dot(ef_ref[...], w1_ref[...], preferred_element_type=jnp.float32)
    hid = jnp.maximum(hid + b1_ref[...], 0.0)
    res = (jnp.dot(hid, w2_ref[...], preferred_element_type=jnp.float32)
           + b2_ref[...])
    r5 = res.reshape(BE // C, 2, C // 2, D)
    for half, o in ((0, 0), (1, DH)):
        packed = jnp.concatenate(
            [r5[:, 0, :, o:o + DH], r5[:, 1, :, o:o + DH]], axis=2)
        out_ref[half] = packed.reshape(BE // 2, D)


def _edge_mlp(ef, w1, b1, w2, b2):
    return pl.pallas_call(
        _edge_mlp_body,
        grid=(E // BE,),
        in_specs=[
            pl.BlockSpec((BE, DE), lambda i: (i, 0)),
            pl.BlockSpec((DE, D), lambda i: (0, 0)),
            pl.BlockSpec((1, D), lambda i: (0, 0)),
            pl.BlockSpec((D, D), lambda i: (0, 0)),
            pl.BlockSpec((1, D), lambda i: (0, 0)),
        ],
        out_specs=pl.BlockSpec((2, BE // 2, D), lambda i: (0, i, 0)),
        out_shape=jax.ShapeDtypeStruct((2, E // 2, D), jnp.float32),
    )(ef, w1, b1, w2, b2)


_mesh = plsc.VectorSubcoreMesh(core_axis_name="c", subcore_axis_name="s")


@functools.partial(
    pl.kernel,
    out_type=jax.ShapeDtypeStruct((NC, NP, DH), jnp.float32),
    mesh=_mesh,
    compiler_params=pltpu.CompilerParams(use_tc_tiling_on_sc=False),
    scratch_types=[
        pltpu.VMEM_SHARED((NP, DH), jnp.float32),
        pltpu.VMEM_SHARED((NP, DH), jnp.float32),
        pltpu.VMEM((SG, C), jnp.int32),
        pltpu.VMEM((SG, C), jnp.int32),
        pltpu.VMEM((SG, C), jnp.int32),
        pltpu.VMEM((SG, C), jnp.int32),
        pltpu.VMEM((C, DH), jnp.float32),
        pltpu.VMEM((C, DH), jnp.float32),
        pltpu.VMEM((C // 2, D), jnp.float32),
        pltpu.VMEM((C // 2, D), jnp.float32),
        pltpu.VMEM((C, DH), jnp.float32),
        pltpu.VMEM((C, DH), jnp.float32),
        pltpu.SemaphoreType.DMA,
        pltpu.SemaphoreType.DMA,
        pltpu.SemaphoreType.DMA,
        pltpu.SemaphoreType.DMA,
        pltpu.SemaphoreType.DMA,
        pltpu.SemaphoreType.DMA,
        pltpu.SemaphoreType.DMA,
        pltpu.SemaphoreType.DMA,
    ],
)
def _sc_aggregate(h_hbm, e_hbm, src_hbm, dst_hbm, z_hbm, out_hbm,
                  h_sh, aggr, s0, d0, s1, d1, r0, r1, e0, e1, m0, m1,
                  g0, g1, es0, es1, ss0, ss1, i0, i1):
    cid = lax.axis_index("c")
    sid = lax.axis_index("s")
    rows = (r0, r1)
    ebuf = (e0, e1)
    mbuf = (m0, m1)
    gsem = (g0, g1)
    esem = (es0, es1)
    ssem = (ss0, ss1)
    sbuf = (s0, s1)
    dbuf = (d0, d1)
    isem = (i0, i1)

    sl = pl.ds(sid * RPT, RPT)
    pltpu.sync_copy(h_hbm.at[cid].at[sl], h_sh.at[sl])
    pltpu.sync_copy(z_hbm.at[sl], aggr.at[sl])
    plsc.subcore_barrier()

    pltpu.sync_copy(src_hbm.at[sid, 0], s0)
    pltpu.sync_copy(dst_hbm.at[sid, 0], d0)
    pltpu.async_copy(src_hbm.at[sid, 1], s1, i1)
    pltpu.async_copy(dst_hbm.at[sid, 1], d1, i1)
    for b in range(2):
        pltpu.async_copy(h_sh.at[s0.at[b]], rows[b], gsem[b])
        pltpu.async_copy(
            e_hbm.at[cid].at[pl.ds(sid * (EPW // 2) + b * (C // 2), C // 2)],
            ebuf[b], esem[b])

    def group_body(g, sidx, didx, snext, dnext, isem_cur, isem_next):
        base = sid * (EPW // 2) + g * SG * (C // 2)

        def pair(p, c1):
            for b in range(2):
                j = p * 2 + b
                pltpu.make_async_copy(h_sh.at[sidx.at[j]], rows[b],
                                      gsem[b]).wait()
                pltpu.make_async_copy(
                    e_hbm.at[cid].at[pl.ds(base + j * (C // 2), C // 2)],
                    ebuf[b], esem[b]).wait()

                @pl.when(p > 0)
                def _():
                    pltpu.make_async_copy(mbuf[b], aggr.at[didx.at[j]],
                                          ssem[b]).wait()

                def quad(i, c2):
                    for u in range(4):
                        r = i * 4 + u
                        for half in range(2):
                            m = half * (C // 2) + r
                            for k in range(DH // 16):
                                s = pl.ds(k * 16, 16)
                                se = pl.ds(half * DH + k * 16, 16)
                                mbuf[b][m, s] = jnp.maximum(
                                    rows[b][m, s] + ebuf[b][r, se], 0.0)
                    return c2

                lax.fori_loop(0, C // 8, quad, 0)

                pltpu.async_copy(mbuf[b], aggr.at[didx.at[j]], ssem[b],
                                 add=True)

                @pl.when(j + 2 < SG)
                def _():
                    pltpu.async_copy(h_sh.at[sidx.at[j + 2]], rows[b],
                                     gsem[b])
                    pltpu.async_copy(
                        e_hbm.at[cid].at[
                            pl.ds(base + (j + 2) * (C // 2), C // 2)],
                        ebuf[b], esem[b])
            return c1

        lax.fori_loop(0, SG // 2, pair, 0)

        for b in range(2):
            pltpu.make_async_copy(mbuf[b], aggr.at[didx.at[SG - 2 + b]],
                                  ssem[b]).wait()

        @pl.when(g + 1 < NG)
        def _():
            pltpu.make_async_copy(src_hbm.at[sid, g + 1], snext,
                                  isem_next).wait()
            pltpu.make_async_copy(dst_hbm.at[sid, g + 1], dnext,
                                  isem_next).wait()
            nbase = base + SG * (C // 2)
            for b in range(2):
                pltpu.async_copy(h_sh.at[snext.at[b]], rows[b], gsem[b])
                pltpu.async_copy(
                    e_hbm.at[cid].at[pl.ds(nbase + b * (C // 2), C // 2)],
                    ebuf[b], esem[b])

        @pl.when(g + 2 < NG)
        def _():
            pltpu.async_copy(src_hbm.at[sid, g + 2], sidx, isem_cur)
            pltpu.async_copy(dst_hbm.at[sid, g + 2], didx, isem_cur)

    def group(g, carry):
        @pl.when(g % 2 == 0)
        def _():
            group_body(g, s0, d0, s1, d1, i0, i1)

        @pl.when(g % 2 == 1)
        def _():
            group_body(g, s1, d1, s0, d0, i1, i0)
        return carry

    lax.fori_loop(0, NG, group, 0)

    plsc.subcore_barrier()
    pltpu.sync_copy(aggr.at[sl], out_hbm.at[cid].at[sl])


BN = 1000


def _apply_body(h_ref, p_ref, w_ref, b_ref, out_ref, out01_ref):
    x = h_ref[...] + jnp.concatenate([p_ref[0], p_ref[1]], axis=1)
    y = jnp.dot(x, w_ref[...], preferred_element_type=jnp.float32)
    y = jnp.maximum(y + b_ref[...], 0.0)
    out_ref[...] = y
    out01_ref[0] = y[:, :DH]
    out01_ref[1] = y[:, DH:]


def _apply_layer(h, parts, w, b):
    return pl.pallas_call(
        _apply_body,
        grid=(N // BN,),
        in_specs=[
            pl.BlockSpec((BN, D), lambda i: (i, 0)),
            pl.BlockSpec((NC, BN, DH), lambda i: (0, i, 0)),
            pl.BlockSpec((D, D), lambda i: (0, 0)),
            pl.BlockSpec((1, D), lambda i: (0, 0)),
        ],
        out_specs=[
            pl.BlockSpec((BN, D), lambda i: (i, 0)),
            pl.BlockSpec((2, BN, DH), lambda i: (0, i, 0)),
        ],
        out_shape=[
            jax.ShapeDtypeStruct((N, D), jnp.float32),
            jax.ShapeDtypeStruct((2, NP, DH), jnp.float32),
        ],
    )(h, parts, w, b)


def kernel(node_feats, edge_feats, edge_index, We1, be1, We2, be2,
           W0, b0, W1, b1):
    e01 = _edge_mlp(edge_feats, We1, be1.reshape(1, D), We2,
                    be2.reshape(1, D))
    src = edge_index[0].reshape(NS, NG, SG, C)
    dst = edge_index[1].reshape(NS, NG, SG, C)
    zeros = jnp.zeros((NP, DH), jnp.float32)

    h = node_feats
    h01 = jnp.pad(
        jnp.stack([node_feats[:, :DH], node_feats[:, DH:]]),
        ((0, 0), (0, NP - N), (0, 0)))
    for (w, b) in ((W0, b0), (W1, b1)):
        parts = _sc_aggregate(h01, e01, src, dst, zeros)
        h, h01 = _apply_layer(h, parts, w, b.reshape(1, D))
    return h


# speedup vs baseline: 1.0375x; 1.0375x over previous
import functools

import jax
import jax.numpy as jnp
from jax import lax
from jax.experimental import pallas as pl
from jax.experimental.pallas import tpu as pltpu
import jax.experimental.pallas.tpu_sc as plsc

N = 10000
E = 320000
D = 128
DE = 16
DH = D // 2

NC = 2
NS = 16
C = 80
SG = 10
NP = 10240
RPT = NP // NS

EA = 102400
EB = E - EA
NGA = EA // (NS * SG * C)
NGB = EB // (NS * SG * C)


BE = 1600


def _edge_mlp_body(ef_ref, w1_ref, b1_ref, w2_ref, b2_ref, out_ref):
    hid = jnp.dot(ef_ref[...], w1_ref[...], preferred_element_type=jnp.float32)
    hid = jnp.maximum(hid + b1_ref[...], 0.0)
    res = (jnp.dot(hid, w2_ref[...], preferred_element_type=jnp.float32)
           + b2_ref[...])
    r5 = res.reshape(BE // C, 2, C // 2, D)
    for half, o in ((0, 0), (1, DH)):
        packed = jnp.concatenate(
            [r5[:, 0, :, o:o + DH], r5[:, 1, :, o:o + DH]], axis=2)
        out_ref[half] = packed.reshape(BE // 2, D)


def _edge_mlp(ef, w1, b1, w2, b2, n_edges, block_off):
    return pl.pallas_call(
        _edge_mlp_body,
        grid=(n_edges // BE,),
        in_specs=[
            pl.BlockSpec((BE, DE), lambda i: (i + block_off, 0)),
            pl.BlockSpec((DE, D), lambda i: (0, 0)),
            pl.BlockSpec((1, D), lambda i: (0, 0)),
            pl.BlockSpec((D, D), lambda i: (0, 0)),
            pl.BlockSpec((1, D), lambda i: (0, 0)),
        ],
        out_specs=pl.BlockSpec((2, BE // 2, D), lambda i: (0, i, 0)),
        out_shape=jax.ShapeDtypeStruct((2, n_edges // 2, D), jnp.float32),
    )(ef, w1, b1, w2, b2)


_mesh = plsc.VectorSubcoreMesh(core_axis_name="c", subcore_axis_name="s")


def _make_sc_aggregate(ng):
    erpw = ng * SG * (C // 2)

    @functools.partial(
        pl.kernel,
        out_type=jax.ShapeDtypeStruct((NC, NP, DH), jnp.float32),
        mesh=_mesh,
        compiler_params=pltpu.CompilerParams(use_tc_tiling_on_sc=False),
        scratch_types=[
            pltpu.VMEM_SHARED((NP, DH), jnp.float32),
            pltpu.VMEM_SHARED((NP, DH), jnp.float32),
            pltpu.VMEM((SG, C), jnp.int32),
            pltpu.VMEM((SG, C), jnp.int32),
            pltpu.VMEM((C, DH), jnp.float32),
            pltpu.VMEM((C, DH), jnp.float32),
            pltpu.VMEM((C // 2, D), jnp.float32),
            pltpu.VMEM((C // 2, D), jnp.float32),
            pltpu.VMEM((C, DH), jnp.float32),
            pltpu.VMEM((C, DH), jnp.float32),
            pltpu.SemaphoreType.DMA,
            pltpu.SemaphoreType.DMA,
            pltpu.SemaphoreType.DMA,
            pltpu.SemaphoreType.DMA,
            pltpu.SemaphoreType.DMA,
            pltpu.SemaphoreType.DMA,
        ],
    )
    def sc_aggregate(h_hbm, e_hbm, src_hbm, dst_hbm, init_hbm, out_hbm,
                     h_sh, aggr, sidx, didx, r0, r1, e0, e1, m0, m1,
                     g0, g1, es0, es1, ss0, ss1):
        cid = lax.axis_index("c")
        sid = lax.axis_index("s")
        rows = (r0, r1)
        ebuf = (e0, e1)
        mbuf = (m0, m1)
        gsem = (g0, g1)
        esem = (es0, es1)
        ssem = (ss0, ss1)

        sl = pl.ds(sid * RPT, RPT)
        pltpu.sync_copy(h_hbm.at[cid].at[sl], h_sh.at[sl])
        pltpu.sync_copy(init_hbm.at[cid].at[sl], aggr.at[sl])
        plsc.subcore_barrier()

        def group(g, carry):
            pltpu.sync_copy(src_hbm.at[sid, g], sidx)
            pltpu.sync_copy(dst_hbm.at[sid, g], didx)
            base = sid * erpw + g * SG * (C // 2)

            for b in range(2):
                pltpu.async_copy(h_sh.at[sidx.at[b]], rows[b], gsem[b])
                pltpu.async_copy(
                    e_hbm.at[cid].at[pl.ds(base + b * (C // 2), C // 2)],
                    ebuf[b], esem[b])

            def pair(p, c1):
                for b in range(2):
                    j = p * 2 + b
                    pltpu.make_async_copy(h_sh.at[sidx.at[j]], rows[b],
                                          gsem[b]).wait()
                    pltpu.make_async_copy(
                        e_hbm.at[cid].at[pl.ds(base + j * (C // 2), C // 2)],
                        ebuf[b], esem[b]).wait()

                    @pl.when(p > 0)
                    def _():
                        pltpu.make_async_copy(mbuf[b], aggr.at[didx.at[j]],
                                              ssem[b]).wait()

                    def quad(i, c2):
                        for u in range(4):
                            r = i * 4 + u
                            for half in range(2):
                                m = half * (C // 2) + r
                                for k in range(DH // 16):
                                    s = pl.ds(k * 16, 16)
                                    se = pl.ds(half * DH + k * 16, 16)
                                    mbuf[b][m, s] = jnp.maximum(
                                        rows[b][m, s] + ebuf[b][r, se], 0.0)
                        return c2

                    lax.fori_loop(0, C // 8, quad, 0)

                    pltpu.async_copy(mbuf[b], aggr.at[didx.at[j]], ssem[b],
                                     add=True)

                    @pl.when(j + 2 < SG)
                    def _():
                        pltpu.async_copy(h_sh.at[sidx.at[j + 2]], rows[b],
                                         gsem[b])
                        pltpu.async_copy(
                            e_hbm.at[cid].at[
                                pl.ds(base + (j + 2) * (C // 2), C // 2)],
                            ebuf[b], esem[b])
                return c1

            lax.fori_loop(0, SG // 2, pair, 0)

            for b in range(2):
                pltpu.make_async_copy(mbuf[b], aggr.at[didx.at[SG - 2 + b]],
                                      ssem[b]).wait()
            return carry

        lax.fori_loop(0, ng, group, 0)

        plsc.subcore_barrier()
        pltpu.sync_copy(aggr.at[sl], out_hbm.at[cid].at[sl])

    return sc_aggregate


_sc_aggr_a = _make_sc_aggregate(NGA)
_sc_aggr_b = _make_sc_aggregate(NGB)


BN = 1000


def _apply_body(h_ref, p_ref, w_ref, b_ref, out_ref, out01_ref):
    x = h_ref[...] + jnp.concatenate([p_ref[0], p_ref[1]], axis=1)
    y = jnp.dot(x, w_ref[...], preferred_element_type=jnp.float32)
    y = jnp.maximum(y + b_ref[...], 0.0)
    out_ref[...] = y
    out01_ref[0] = y[:, :DH]
    out01_ref[1] = y[:, DH:]


def _apply_layer(h, parts, w, b):
    return pl.pallas_call(
        _apply_body,
        grid=(N // BN,),
        in_specs=[
            pl.BlockSpec((BN, D), lambda i: (i, 0)),
            pl.BlockSpec((NC, BN, DH), lambda i: (0, i, 0)),
            pl.BlockSpec((D, D), lambda i: (0, 0)),
            pl.BlockSpec((1, D), lambda i: (0, 0)),
        ],
        out_specs=[
            pl.BlockSpec((BN, D), lambda i: (i, 0)),
            pl.BlockSpec((2, BN, DH), lambda i: (0, i, 0)),
        ],
        out_shape=[
            jax.ShapeDtypeStruct((N, D), jnp.float32),
            jax.ShapeDtypeStruct((2, NP, DH), jnp.float32),
        ],
    )(h, parts, w, b)


def kernel(node_feats, edge_feats, edge_index, We1, be1, We2, be2,
           W0, b0, W1, b1):
    b1e = be1.reshape(1, D)
    b2e = be2.reshape(1, D)
    srcA = edge_index[0, :EA].reshape(NS, NGA, SG, C)
    dstA = edge_index[1, :EA].reshape(NS, NGA, SG, C)
    srcB = edge_index[0, EA:].reshape(NS, NGB, SG, C)
    dstB = edge_index[1, EA:].reshape(NS, NGB, SG, C)
    zeros = jnp.zeros((NC, NP, DH), jnp.float32)

    h = node_feats
    h01 = jnp.pad(
        jnp.stack([node_feats[:, :DH], node_feats[:, DH:]]),
        ((0, 0), (0, NP - N), (0, 0)))

    eA = _edge_mlp(edge_feats, We1, b1e, We2, b2e, EA, 0)
    pA0 = _sc_aggr_a(h01, eA, srcA, dstA, zeros)
    eB = _edge_mlp(edge_feats, We1, b1e, We2, b2e, EB, EA // BE)
    parts0 = _sc_aggr_b(h01, eB, srcB, dstB, pA0)
    h, h01 = _apply_layer(h, parts0, W0, b0.reshape(1, D))

    pA1 = _sc_aggr_a(h01, eA, srcA, dstA, zeros)
    parts1 = _sc_aggr_b(h01, eB, srcB, dstB, pA1)
    h, _ = _apply_layer(h, parts1, W1, b1.reshape(1, D))
    return h


# speedup vs baseline: 1.5693x; 1.5126x over previous
import functools

import jax
import jax.numpy as jnp
from jax import lax
from jax.experimental import pallas as pl
from jax.experimental.pallas import tpu as pltpu
import jax.experimental.pallas.tpu_sc as plsc

N = 10000
E = 320000
D = 128
DE = 16
DH = D // 2

NC = 2
NS = 16
C = 80
SG = 10
NP = 10240
RPT = NP // NS

EA = 102400
EB = E - EA
NGA = EA // (NS * SG * C)
NGB = EB // (NS * SG * C)


BE = 1600


def _edge_mlp_body(ef_ref, w1_ref, b1_ref, w2_ref, b2_ref, out_ref):
    hid = jnp.dot(ef_ref[...], w1_ref[...], preferred_element_type=jnp.float32)
    hid = jnp.maximum(hid + b1_ref[...], 0.0)
    res = (jnp.dot(hid, w2_ref[...], preferred_element_type=jnp.float32)
           + b2_ref[...])
    r5 = res.reshape(BE // C, 2, C // 2, D)
    for half, o in ((0, 0), (1, DH)):
        packed = jnp.concatenate(
            [r5[:, 0, :, o:o + DH], r5[:, 1, :, o:o + DH]], axis=2)
        out_ref[half] = packed.reshape(BE // 2, D)


def _edge_mlp(ef, w1, b1, w2, b2, n_edges, block_off):
    return pl.pallas_call(
        _edge_mlp_body,
        grid=(n_edges // BE,),
        in_specs=[
            pl.BlockSpec((BE, DE), lambda i: (i + block_off, 0)),
            pl.BlockSpec((DE, D), lambda i: (0, 0)),
            pl.BlockSpec((1, D), lambda i: (0, 0)),
            pl.BlockSpec((D, D), lambda i: (0, 0)),
            pl.BlockSpec((1, D), lambda i: (0, 0)),
        ],
        out_specs=pl.BlockSpec((2, BE // 2, D), lambda i: (0, i, 0)),
        out_shape=jax.ShapeDtypeStruct((2, n_edges // 2, D), jnp.float32),
    )(ef, w1, b1, w2, b2)


_mesh = plsc.VectorSubcoreMesh(core_axis_name="c", subcore_axis_name="s")


def _make_sc_aggregate(ng):
    erpw = ng * SG * (C // 2)

    @functools.partial(
        pl.kernel,
        out_type=jax.ShapeDtypeStruct((NC, NP, DH), jnp.float32),
        mesh=_mesh,
        compiler_params=pltpu.CompilerParams(use_tc_tiling_on_sc=False),
        scratch_types=[
            pltpu.VMEM_SHARED((NP, DH), jnp.float32),
            pltpu.VMEM_SHARED((NP, DH), jnp.float32),
            pltpu.VMEM((SG, C), jnp.int32),
            pltpu.VMEM((SG, C), jnp.int32),
            pltpu.VMEM((C, DH), jnp.float32),
            pltpu.VMEM((C, DH), jnp.float32),
            pltpu.VMEM((C // 2, D), jnp.float32),
            pltpu.VMEM((C // 2, D), jnp.float32),
            pltpu.VMEM((C, DH), jnp.float32),
            pltpu.VMEM((C, DH), jnp.float32),
            pltpu.SemaphoreType.DMA,
            pltpu.SemaphoreType.DMA,
            pltpu.SemaphoreType.DMA,
            pltpu.SemaphoreType.DMA,
            pltpu.SemaphoreType.DMA,
            pltpu.SemaphoreType.DMA,
        ],
    )
    def sc_aggregate(h_hbm, e_hbm, src_hbm, dst_hbm, init_hbm, out_hbm,
                     h_sh, aggr, sidx, didx, r0, r1, e0, e1, m0, m1,
                     g0, g1, es0, es1, ss0, ss1):
        cid = lax.axis_index("c")
        sid = lax.axis_index("s")
        rows = (r0, r1)
        ebuf = (e0, e1)
        mbuf = (m0, m1)
        gsem = (g0, g1)
        esem = (es0, es1)
        ssem = (ss0, ss1)

        sl = pl.ds(sid * RPT, RPT)
        pltpu.sync_copy(h_hbm.at[cid].at[sl], h_sh.at[sl])
        pltpu.sync_copy(init_hbm.at[cid].at[sl], aggr.at[sl])
        plsc.subcore_barrier()

        def group(g, carry):
            pltpu.sync_copy(src_hbm.at[sid, g], sidx)
            pltpu.sync_copy(dst_hbm.at[sid, g], didx)
            base = sid * erpw + g * SG * (C // 2)

            for b in range(2):
                pltpu.async_copy(h_sh.at[sidx.at[b]], rows[b], gsem[b])
                pltpu.async_copy(
                    e_hbm.at[cid].at[pl.ds(base + b * (C // 2), C // 2)],
                    ebuf[b], esem[b])

            def pair(p, c1):
                for b in range(2):
                    j = p * 2 + b
                    pltpu.make_async_copy(h_sh.at[sidx.at[j]], rows[b],
                                          gsem[b]).wait()
                    pltpu.make_async_copy(
                        e_hbm.at[cid].at[pl.ds(base + j * (C // 2), C // 2)],
                        ebuf[b], esem[b]).wait()

                    @pl.when(p > 0)
                    def _():
                        pltpu.make_async_copy(mbuf[b], aggr.at[didx.at[j]],
                                              ssem[b]).wait()

                    @plsc.parallel_loop(0, C // 2, step=4)
                    def _(r0_):
                        for u in range(4):
                            r = r0_ + u
                            for k in range(DH // 16):
                                s = pl.ds(k * 16, 16)
                                mbuf[b][r, s] = jnp.maximum(
                                    rows[b][r, s] + ebuf[b][r, s], 0.0)

                    @plsc.parallel_loop(0, C // 2, step=4)
                    def _(r0_):
                        for u in range(4):
                            r = r0_ + u
                            m = r + C // 2
                            for k in range(DH // 16):
                                s = pl.ds(k * 16, 16)
                                se = pl.ds(DH + k * 16, 16)
                                mbuf[b][m, s] = jnp.maximum(
                                    rows[b][m, s] + ebuf[b][r, se], 0.0)

                    pltpu.async_copy(mbuf[b], aggr.at[didx.at[j]], ssem[b],
                                     add=True)

                    @pl.when(j + 2 < SG)
                    def _():
                        pltpu.async_copy(h_sh.at[sidx.at[j + 2]], rows[b],
                                         gsem[b])
                        pltpu.async_copy(
                            e_hbm.at[cid].at[
                                pl.ds(base + (j + 2) * (C // 2), C // 2)],
                            ebuf[b], esem[b])
                return c1

            lax.fori_loop(0, SG // 2, pair, 0)

            for b in range(2):
                pltpu.make_async_copy(mbuf[b], aggr.at[didx.at[SG - 2 + b]],
                                      ssem[b]).wait()
            return carry

        lax.fori_loop(0, ng, group, 0)

        plsc.subcore_barrier()
        pltpu.sync_copy(aggr.at[sl], out_hbm.at[cid].at[sl])

    return sc_aggregate


_sc_aggr_a = _make_sc_aggregate(NGA)
_sc_aggr_b = _make_sc_aggregate(NGB)


BN = 1000


def _apply_body(h_ref, p_ref, w_ref, b_ref, out_ref, out01_ref):
    x = h_ref[...] + jnp.concatenate([p_ref[0], p_ref[1]], axis=1)
    y = jnp.dot(x, w_ref[...], preferred_element_type=jnp.float32)
    y = jnp.maximum(y + b_ref[...], 0.0)
    out_ref[...] = y
    out01_ref[0] = y[:, :DH]
    out01_ref[1] = y[:, DH:]


def _apply_layer(h, parts, w, b):
    return pl.pallas_call(
        _apply_body,
        grid=(N // BN,),
        in_specs=[
            pl.BlockSpec((BN, D), lambda i: (i, 0)),
            pl.BlockSpec((NC, BN, DH), lambda i: (0, i, 0)),
            pl.BlockSpec((D, D), lambda i: (0, 0)),
            pl.BlockSpec((1, D), lambda i: (0, 0)),
        ],
        out_specs=[
            pl.BlockSpec((BN, D), lambda i: (i, 0)),
            pl.BlockSpec((2, BN, DH), lambda i: (0, i, 0)),
        ],
        out_shape=[
            jax.ShapeDtypeStruct((N, D), jnp.float32),
            jax.ShapeDtypeStruct((2, NP, DH), jnp.float32),
        ],
    )(h, parts, w, b)


def kernel(node_feats, edge_feats, edge_index, We1, be1, We2, be2,
           W0, b0, W1, b1):
    b1e = be1.reshape(1, D)
    b2e = be2.reshape(1, D)
    srcA = edge_index[0, :EA].reshape(NS, NGA, SG, C)
    dstA = edge_index[1, :EA].reshape(NS, NGA, SG, C)
    srcB = edge_index[0, EA:].reshape(NS, NGB, SG, C)
    dstB = edge_index[1, EA:].reshape(NS, NGB, SG, C)
    zeros = jnp.zeros((NC, NP, DH), jnp.float32)

    h = node_feats
    h01 = jnp.pad(
        jnp.stack([node_feats[:, :DH], node_feats[:, DH:]]),
        ((0, 0), (0, NP - N), (0, 0)))

    eA = _edge_mlp(edge_feats, We1, b1e, We2, b2e, EA, 0)
    pA0 = _sc_aggr_a(h01, eA, srcA, dstA, zeros)
    eB = _edge_mlp(edge_feats, We1, b1e, We2, b2e, EB, EA // BE)
    parts0 = _sc_aggr_b(h01, eB, srcB, dstB, pA0)
    h, h01 = _apply_layer(h, parts0, W0, b0.reshape(1, D))

    pA1 = _sc_aggr_a(h01, eA, srcA, dstA, zeros)
    parts1 = _sc_aggr_b(h01, eB, srcB, dstB, pA1)
    h, _ = _apply_layer(h, parts1, W1, b1.reshape(1, D))
    return h
